# Initial kernel scaffold; baseline (speedup 1.0000x reference)
import jax
import jax.numpy as jnp
from jax.experimental import pallas as pl


def kernel(indices, output, label, stored_targets):
    raise NotImplementedError("write your pallas kernel here")


# speedup vs baseline: 2.9315x; 2.9315x over previous
import functools

import jax
import jax.numpy as jnp
from jax import lax
from jax.experimental import pallas as pl
from jax.experimental.pallas import tpu as pltpu
from jax.experimental.pallas import tpu_sc as plsc

N_EXAMPLES = 100000
N_CLASSES = 128
BATCH = 4096
BETA = 0.3
LAM = 3.0

NC = 2
NS = 16
NW = NC * NS
ROWS_PER_TILE = BATCH // NW
SUB = 32
NSUB = ROWS_PER_TILE // SUB
NCHUNK = BATCH // 16


def _stage1_body(out_ref, label_ref, pred_ref, pn_ref, ce_ref):
    x = out_ref[...]
    m = jnp.max(x, axis=1, keepdims=True)
    e = jnp.exp(x - m)
    se = jnp.sum(e, axis=1, keepdims=True)
    logp = (x - m) - jnp.log(se)
    pred = jnp.clip(e / se, 0.0001, 1.0 - 0.0001)
    pn = pred / jnp.sum(pred, axis=1, keepdims=True)
    pred_ref[...] = pred
    pn_ref[...] = pn
    lab = label_ref[...]
    cols = lax.broadcasted_iota(jnp.int32, x.shape, 1)
    sel = jnp.where(cols == lab, logp, 0.0)
    ce_ref[...] = jnp.reshape(-jnp.sum(sel) / BATCH, (1, 1))


_stage1 = pl.pallas_call(
    _stage1_body,
    out_shape=(
        jax.ShapeDtypeStruct((BATCH, N_CLASSES), jnp.float32),
        jax.ShapeDtypeStruct((BATCH, N_CLASSES), jnp.float32),
        jax.ShapeDtypeStruct((1, 1), jnp.float32),
    ),
)


def _sc_body(idx_hbm, pn_hbm, stored_hbm, pn_rows_hbm, st_rows_hbm,
             table, idxv, myidx, wv, pnbuf, stbuf, sem):
    wid = lax.axis_index("s") * NC + lax.axis_index("c")
    base = wid * ROWS_PER_TILE

    pltpu.sync_copy(idx_hbm, idxv)

    def chunk_body(c, carry):
        off = pl.multiple_of(c * 16, 16)
        keys = idxv[pl.ds(off, 16)]
        pos = jnp.full((16,), c * 16, jnp.int32) + lax.iota(jnp.int32, 16)
        _, last = plsc.scan_count(keys)
        plsc.store_scatter(table, [keys], pos, mask=last)
        return carry

    lax.fori_loop(0, NCHUNK, chunk_body, 0)

    for c in range(ROWS_PER_TILE // 16):
        keys = idxv[pl.ds(base + c * 16, 16)]
        w = plsc.load_gather(table, [keys])
        b, r = (c * 16) // SUB, (c * 16) % SUB
        myidx[b, pl.ds(r, 16)] = keys
        wv[b, pl.ds(r, 16)] = w

    for b in range(NSUB):
        pltpu.async_copy(pn_hbm.at[wv.at[b]], pnbuf, sem).wait()
        pltpu.sync_copy(pnbuf, pn_rows_hbm.at[pl.ds(base + b * SUB, SUB)])
        pltpu.async_copy(stored_hbm.at[myidx.at[b]], stbuf, sem).wait()
        pltpu.sync_copy(stbuf, st_rows_hbm.at[pl.ds(base + b * SUB, SUB)])


_stage2 = functools.partial(
    pl.kernel,
    out_type=(
        jax.ShapeDtypeStruct((BATCH, N_CLASSES), jnp.float32),
        jax.ShapeDtypeStruct((BATCH, N_CLASSES), jnp.float32),
    ),
    mesh=plsc.VectorSubcoreMesh(core_axis_name="c", subcore_axis_name="s"),
    compiler_params=pltpu.CompilerParams(needs_layout_passes=False),
    scratch_types=(
        pltpu.VMEM((N_EXAMPLES,), jnp.int32),
        pltpu.VMEM((BATCH,), jnp.int32),
        pltpu.VMEM((NSUB, SUB), jnp.int32),
        pltpu.VMEM((NSUB, SUB), jnp.int32),
        pltpu.VMEM((SUB, N_CLASSES), jnp.float32),
        pltpu.VMEM((SUB, N_CLASSES), jnp.float32),
        pltpu.SemaphoreType.DMA,
    ),
)(_sc_body)


def _stage3_body(pred_ref, pn_rows_ref, st_rows_ref, ce_ref, out_ref):
    pred = pred_ref[...]
    g = BETA * st_rows_ref[...] + (1.0 - BETA) * pn_rows_ref[...]
    s = jnp.sum(g * pred, axis=1)
    term = jnp.log(1.0 - s)
    out_ref[...] = ce_ref[...] + LAM * jnp.reshape(jnp.mean(term), (1, 1))


_stage3 = pl.pallas_call(
    _stage3_body,
    out_shape=jax.ShapeDtypeStruct((1, 1), jnp.float32),
)


def kernel(indices, output, label, stored_targets):
    label2 = label.reshape(BATCH, 1).astype(jnp.int32)
    pred, pn, ce = _stage1(output, label2)
    pn_rows, st_rows = _stage2(indices, pn, stored_targets)
    loss = _stage3(pred, pn_rows, st_rows, ce)
    return loss.reshape(())


# speedup vs baseline: 3.3093x; 1.1289x over previous
import functools

import jax
import jax.numpy as jnp
from jax import lax
from jax.experimental import pallas as pl
from jax.experimental.pallas import tpu as pltpu
from jax.experimental.pallas import tpu_sc as plsc

N_EXAMPLES = 100000
N_CLASSES = 128
BATCH = 4096
BETA = 0.3
LAM = 3.0

NC = 2
NS = 16
NW = NC * NS
ROWS_PER_TILE = BATCH // NW
SUB = 32
NSUB = ROWS_PER_TILE // SUB
NCHUNK = BATCH // 16


def _stage1_body(out_ref, label_ref, pred_ref, pn_ref, ce_ref):
    x = out_ref[...]
    m = jnp.max(x, axis=1, keepdims=True)
    e = jnp.exp(x - m)
    se = jnp.sum(e, axis=1, keepdims=True)
    logp = (x - m) - jnp.log(se)
    pred = jnp.clip(e / se, 0.0001, 1.0 - 0.0001)
    pn = pred / jnp.sum(pred, axis=1, keepdims=True)
    pred_ref[...] = pred
    pn_ref[...] = pn
    lab = label_ref[...]
    cols = lax.broadcasted_iota(jnp.int32, x.shape, 1)
    sel = jnp.where(cols == lab, logp, 0.0)
    ce_ref[...] = jnp.reshape(-jnp.sum(sel) / BATCH, (1, 1))


_stage1 = pl.pallas_call(
    _stage1_body,
    out_shape=(
        jax.ShapeDtypeStruct((BATCH, N_CLASSES), jnp.float32),
        jax.ShapeDtypeStruct((BATCH, N_CLASSES), jnp.float32),
        jax.ShapeDtypeStruct((1, 1), jnp.float32),
    ),
)


UNROLL = 4


def _sc_body(idx_hbm, pn_hbm, stored_hbm, pn_rows_hbm, st_rows_hbm,
             table, idxv, myidx, wv, pnbuf, stbuf,
             gp0, gp1, gs0, gs1, wp0, wp1, ws0, ws1):
    wid = lax.axis_index("s") * NC + lax.axis_index("c")
    base = wid * ROWS_PER_TILE

    pltpu.sync_copy(idx_hbm, idxv)

    def chunk_body(i, carry):
        for u in range(UNROLL):
            c = i * UNROLL + u
            off = pl.multiple_of(c * 16, 16)
            keys = idxv[pl.ds(off, 16)]
            pos = jnp.full((16,), c * 16, jnp.int32) + lax.iota(jnp.int32, 16)
            _, last = plsc.scan_count(keys)
            plsc.store_scatter(table, [keys], pos, mask=last)
        return carry

    lax.fori_loop(0, NCHUNK // UNROLL, chunk_body, 0)

    for c in range(ROWS_PER_TILE // 16):
        keys = idxv[pl.ds(base + c * 16, 16)]
        w = plsc.load_gather(table, [keys])
        b, r = (c * 16) // SUB, (c * 16) % SUB
        myidx[b, pl.ds(r, 16)] = keys
        wv[b, pl.ds(r, 16)] = w

    gsem = (gp0, gp1, gs0, gs1)
    wsem = (wp0, wp1, ws0, ws1)
    gathers = [None, None]
    writes = [None, None]

    def start(b):
        k = b % 2
        g0 = pltpu.async_copy(pn_hbm.at[wv.at[b]], pnbuf.at[k], gsem[k])
        g1 = pltpu.async_copy(stored_hbm.at[myidx.at[b]], stbuf.at[k],
                              gsem[2 + k])
        gathers[k] = (g0, g1)

    start(0)
    for b in range(NSUB):
        if b + 1 < NSUB:
            if writes[(b + 1) % 2] is not None:
                for wcp in writes[(b + 1) % 2]:
                    wcp.wait()
                writes[(b + 1) % 2] = None
            start(b + 1)
        k = b % 2
        for gcp in gathers[k]:
            gcp.wait()
        dst = pl.ds(base + b * SUB, SUB)
        w0 = pltpu.async_copy(pnbuf.at[k], pn_rows_hbm.at[dst], wsem[k])
        w1 = pltpu.async_copy(stbuf.at[k], st_rows_hbm.at[dst], wsem[2 + k])
        writes[k] = (w0, w1)
    for pair in writes:
        if pair is not None:
            for wcp in pair:
                wcp.wait()


_stage2 = functools.partial(
    pl.kernel,
    out_type=(
        jax.ShapeDtypeStruct((BATCH, N_CLASSES), jnp.float32),
        jax.ShapeDtypeStruct((BATCH, N_CLASSES), jnp.float32),
    ),
    mesh=plsc.VectorSubcoreMesh(core_axis_name="c", subcore_axis_name="s"),
    compiler_params=pltpu.CompilerParams(needs_layout_passes=False),
    scratch_types=(
        pltpu.VMEM((N_EXAMPLES,), jnp.int32),
        pltpu.VMEM((BATCH,), jnp.int32),
        pltpu.VMEM((NSUB, SUB), jnp.int32),
        pltpu.VMEM((NSUB, SUB), jnp.int32),
        pltpu.VMEM((2, SUB, N_CLASSES), jnp.float32),
        pltpu.VMEM((2, SUB, N_CLASSES), jnp.float32),
        pltpu.SemaphoreType.DMA,
        pltpu.SemaphoreType.DMA,
        pltpu.SemaphoreType.DMA,
        pltpu.SemaphoreType.DMA,
        pltpu.SemaphoreType.DMA,
        pltpu.SemaphoreType.DMA,
        pltpu.SemaphoreType.DMA,
        pltpu.SemaphoreType.DMA,
    ),
)(_sc_body)


def _stage3_body(pred_ref, pn_rows_ref, st_rows_ref, ce_ref, out_ref):
    pred = pred_ref[...]
    g = BETA * st_rows_ref[...] + (1.0 - BETA) * pn_rows_ref[...]
    s = jnp.sum(g * pred, axis=1)
    term = jnp.log(1.0 - s)
    out_ref[...] = ce_ref[...] + LAM * jnp.reshape(jnp.mean(term), (1, 1))


_stage3 = pl.pallas_call(
    _stage3_body,
    out_shape=jax.ShapeDtypeStruct((1, 1), jnp.float32),
)


def kernel(indices, output, label, stored_targets):
    label2 = label.reshape(BATCH, 1).astype(jnp.int32)
    pred, pn, ce = _stage1(output, label2)
    pn_rows, st_rows = _stage2(indices, pn, stored_targets)
    loss = _stage3(pred, pn_rows, st_rows, ce)
    return loss.reshape(())


# speedup vs baseline: 3.3159x; 1.0020x over previous
import functools

import jax
import jax.numpy as jnp
from jax import lax
from jax.experimental import pallas as pl
from jax.experimental.pallas import tpu as pltpu
from jax.experimental.pallas import tpu_sc as plsc

N_EXAMPLES = 100000
N_CLASSES = 128
BATCH = 4096
BETA = 0.3
LAM = 3.0

NC = 2
NS = 16
NW = NC * NS
ROWS_PER_TILE = BATCH // NW
SUB = 32
NSUB = ROWS_PER_TILE // SUB
NCHUNK = BATCH // 16


def _stage1_body(out_ref, label_ref, pred_ref, pn_ref, ce_ref):
    x = out_ref[...]
    m = jnp.max(x, axis=1, keepdims=True)
    e = jnp.exp(x - m)
    se = jnp.sum(e, axis=1, keepdims=True)
    logp = (x - m) - jnp.log(se)
    pred = jnp.clip(e * (1.0 / se), 0.0001, 1.0 - 0.0001)
    pn = pred * (1.0 / jnp.sum(pred, axis=1, keepdims=True))
    pred_ref[...] = pred
    pn_ref[...] = pn
    lab = label_ref[...]
    cols = lax.broadcasted_iota(jnp.int32, x.shape, 1)
    sel = jnp.where(cols == lab, logp, 0.0)
    ce_ref[...] = jnp.reshape(-jnp.sum(sel) / BATCH, (1, 1))


_stage1 = pl.pallas_call(
    _stage1_body,
    out_shape=(
        jax.ShapeDtypeStruct((BATCH, N_CLASSES), jnp.float32),
        jax.ShapeDtypeStruct((BATCH, N_CLASSES), jnp.float32),
        jax.ShapeDtypeStruct((1, 1), jnp.float32),
    ),
)


UNROLL = 4


def _sc_body(idx_hbm, pn_hbm, stored_hbm, pn_rows_hbm, st_rows_hbm,
             table, idxv, myidx, wv, pnbuf, stbuf,
             gp0, gp1, gs0, gs1, wp0, wp1, ws0, ws1):
    wid = lax.axis_index("s") * NC + lax.axis_index("c")
    base = wid * ROWS_PER_TILE

    pltpu.sync_copy(idx_hbm, idxv)

    def chunk_body(i, carry):
        for u in range(UNROLL):
            c = i * UNROLL + u
            off = pl.multiple_of(c * 16, 16)
            keys = idxv[pl.ds(off, 16)]
            pos = jnp.full((16,), c * 16, jnp.int32) + lax.iota(jnp.int32, 16)
            _, last = plsc.scan_count(keys)
            plsc.store_scatter(table, [keys], pos, mask=last)
        return carry

    lax.fori_loop(0, NCHUNK // UNROLL, chunk_body, 0)

    for c in range(ROWS_PER_TILE // 16):
        keys = idxv[pl.ds(base + c * 16, 16)]
        w = plsc.load_gather(table, [keys])
        b, r = (c * 16) // SUB, (c * 16) % SUB
        myidx[b, pl.ds(r, 16)] = keys
        wv[b, pl.ds(r, 16)] = w

    gsem = (gp0, gp1, gs0, gs1)
    wsem = (wp0, wp1, ws0, ws1)
    gathers = [None, None]
    writes = [None, None]

    def start(b):
        k = b % 2
        g0 = pltpu.async_copy(pn_hbm.at[wv.at[b]], pnbuf.at[k], gsem[k])
        g1 = pltpu.async_copy(stored_hbm.at[myidx.at[b]], stbuf.at[k],
                              gsem[2 + k])
        gathers[k] = (g0, g1)

    start(0)
    for b in range(NSUB):
        if b + 1 < NSUB:
            if writes[(b + 1) % 2] is not None:
                for wcp in writes[(b + 1) % 2]:
                    wcp.wait()
                writes[(b + 1) % 2] = None
            start(b + 1)
        k = b % 2
        for gcp in gathers[k]:
            gcp.wait()
        dst = pl.ds(base + b * SUB, SUB)
        w0 = pltpu.async_copy(pnbuf.at[k], pn_rows_hbm.at[dst], wsem[k])
        w1 = pltpu.async_copy(stbuf.at[k], st_rows_hbm.at[dst], wsem[2 + k])
        writes[k] = (w0, w1)
    for pair in writes:
        if pair is not None:
            for wcp in pair:
                wcp.wait()


_stage2 = functools.partial(
    pl.kernel,
    out_type=(
        jax.ShapeDtypeStruct((BATCH, N_CLASSES), jnp.float32),
        jax.ShapeDtypeStruct((BATCH, N_CLASSES), jnp.float32),
    ),
    mesh=plsc.VectorSubcoreMesh(core_axis_name="c", subcore_axis_name="s"),
    compiler_params=pltpu.CompilerParams(needs_layout_passes=False,
                                         use_tc_tiling_on_sc=True),
    scratch_types=(
        pltpu.VMEM((N_EXAMPLES,), jnp.int32),
        pltpu.VMEM((BATCH,), jnp.int32),
        pltpu.VMEM((NSUB, SUB), jnp.int32),
        pltpu.VMEM((NSUB, SUB), jnp.int32),
        pltpu.VMEM((2, SUB, N_CLASSES), jnp.float32),
        pltpu.VMEM((2, SUB, N_CLASSES), jnp.float32),
        pltpu.SemaphoreType.DMA,
        pltpu.SemaphoreType.DMA,
        pltpu.SemaphoreType.DMA,
        pltpu.SemaphoreType.DMA,
        pltpu.SemaphoreType.DMA,
        pltpu.SemaphoreType.DMA,
        pltpu.SemaphoreType.DMA,
        pltpu.SemaphoreType.DMA,
    ),
)(_sc_body)


def _stage3_body(pred_ref, pn_rows_ref, st_rows_ref, ce_ref, out_ref):
    pred = pred_ref[...]
    g = BETA * st_rows_ref[...] + (1.0 - BETA) * pn_rows_ref[...]
    s = jnp.sum(g * pred, axis=1)
    term = jnp.log(1.0 - s)
    out_ref[...] = ce_ref[...] + LAM * jnp.reshape(jnp.mean(term), (1, 1))


_stage3 = pl.pallas_call(
    _stage3_body,
    out_shape=jax.ShapeDtypeStruct((1, 1), jnp.float32),
)


def kernel(indices, output, label, stored_targets):
    label2 = label.reshape(BATCH, 1).astype(jnp.int32)
    pred, pn, ce = _stage1(output, label2)
    pn_rows, st_rows = _stage2(indices, pn, stored_targets)
    loss = _stage3(pred, pn_rows, st_rows, ce)
    return loss.reshape(())


# speedup vs baseline: 3.5667x; 1.0756x over previous
import functools

import jax
import jax.numpy as jnp
from jax import lax
from jax.experimental import pallas as pl
from jax.experimental.pallas import tpu as pltpu
from jax.experimental.pallas import tpu_sc as plsc

N_EXAMPLES = 100000
N_CLASSES = 128
BATCH = 4096
BETA = 0.3
LAM = 3.0

NC = 2
NS = 16
NW = NC * NS
ROWS_PER_TILE = BATCH // NW
SUB = 32
NSUB = ROWS_PER_TILE // SUB
NCHUNK = BATCH // 16


def _stage1_body(out_ref, label_ref, pred_ref, ce_ref):
    x = out_ref[...]
    m = jnp.max(x, axis=1, keepdims=True)
    e = jnp.exp(x - m)
    se = jnp.sum(e, axis=1, keepdims=True)
    logp = (x - m) - jnp.log(se)
    pred = jnp.clip(e * (1.0 / se), 0.0001, 1.0 - 0.0001)
    pred_ref[...] = pred
    lab = label_ref[...]
    cols = lax.broadcasted_iota(jnp.int32, x.shape, 1)
    sel = jnp.where(cols == lab, logp, 0.0)
    ce_ref[...] = jnp.reshape(-jnp.sum(sel) / BATCH, (1, 1))


_stage1 = pl.pallas_call(
    _stage1_body,
    out_shape=(
        jax.ShapeDtypeStruct((BATCH, N_CLASSES), jnp.float32),
        jax.ShapeDtypeStruct((1, 1), jnp.float32),
    ),
)


UNROLL = 4


def _sc_body(idx_hbm, logits_hbm, stored_hbm, lo_rows_hbm, st_rows_hbm,
             table, idxv, myidx, wv, pnbuf, stbuf,
             gp0, gp1, gs0, gs1, wp0, wp1, ws0, ws1):
    wid = lax.axis_index("s") * NC + lax.axis_index("c")
    base = wid * ROWS_PER_TILE

    pltpu.sync_copy(idx_hbm, idxv)

    def chunk_body(i, carry):
        for u in range(UNROLL):
            c = i * UNROLL + u
            off = pl.multiple_of(c * 16, 16)
            keys = idxv[pl.ds(off, 16)]
            pos = jnp.full((16,), c * 16, jnp.int32) + lax.iota(jnp.int32, 16)
            _, last = plsc.scan_count(keys)
            plsc.store_scatter(table, [keys], pos, mask=last)
        return carry

    lax.fori_loop(0, NCHUNK // UNROLL, chunk_body, 0)

    for c in range(ROWS_PER_TILE // 16):
        keys = idxv[pl.ds(base + c * 16, 16)]
        w = plsc.load_gather(table, [keys])
        b, r = (c * 16) // SUB, (c * 16) % SUB
        myidx[b, pl.ds(r, 16)] = keys
        wv[b, pl.ds(r, 16)] = w

    gsem = (gp0, gp1, gs0, gs1)
    wsem = (wp0, wp1, ws0, ws1)
    gathers = [None, None]
    writes = [None, None]

    def start(b):
        k = b % 2
        g0 = pltpu.async_copy(logits_hbm.at[wv.at[b]], pnbuf.at[k], gsem[k])
        g1 = pltpu.async_copy(stored_hbm.at[myidx.at[b]], stbuf.at[k],
                              gsem[2 + k])
        gathers[k] = (g0, g1)

    start(0)
    for b in range(NSUB):
        if b + 1 < NSUB:
            if writes[(b + 1) % 2] is not None:
                for wcp in writes[(b + 1) % 2]:
                    wcp.wait()
                writes[(b + 1) % 2] = None
            start(b + 1)
        k = b % 2
        for gcp in gathers[k]:
            gcp.wait()
        dst = pl.ds(base + b * SUB, SUB)
        w0 = pltpu.async_copy(pnbuf.at[k], lo_rows_hbm.at[dst], wsem[k])
        w1 = pltpu.async_copy(stbuf.at[k], st_rows_hbm.at[dst], wsem[2 + k])
        writes[k] = (w0, w1)
    for pair in writes:
        if pair is not None:
            for wcp in pair:
                wcp.wait()


_stage2 = functools.partial(
    pl.kernel,
    out_type=(
        jax.ShapeDtypeStruct((BATCH, N_CLASSES), jnp.float32),
        jax.ShapeDtypeStruct((BATCH, N_CLASSES), jnp.float32),
    ),
    mesh=plsc.VectorSubcoreMesh(core_axis_name="c", subcore_axis_name="s"),
    compiler_params=pltpu.CompilerParams(needs_layout_passes=False,
                                         use_tc_tiling_on_sc=True),
    scratch_types=(
        pltpu.VMEM((N_EXAMPLES,), jnp.int32),
        pltpu.VMEM((BATCH,), jnp.int32),
        pltpu.VMEM((NSUB, SUB), jnp.int32),
        pltpu.VMEM((NSUB, SUB), jnp.int32),
        pltpu.VMEM((2, SUB, N_CLASSES), jnp.float32),
        pltpu.VMEM((2, SUB, N_CLASSES), jnp.float32),
        pltpu.SemaphoreType.DMA,
        pltpu.SemaphoreType.DMA,
        pltpu.SemaphoreType.DMA,
        pltpu.SemaphoreType.DMA,
        pltpu.SemaphoreType.DMA,
        pltpu.SemaphoreType.DMA,
        pltpu.SemaphoreType.DMA,
        pltpu.SemaphoreType.DMA,
    ),
)(_sc_body)


def _stage3_body(pred_ref, lo_rows_ref, st_rows_ref, ce_ref, out_ref):
    pred = pred_ref[...]
    x = lo_rows_ref[...]
    m = jnp.max(x, axis=1, keepdims=True)
    e = jnp.exp(x - m)
    pw = jnp.clip(e * (1.0 / jnp.sum(e, axis=1, keepdims=True)),
                  0.0001, 1.0 - 0.0001)
    pn_rows = pw * (1.0 / jnp.sum(pw, axis=1, keepdims=True))
    g = BETA * st_rows_ref[...] + (1.0 - BETA) * pn_rows
    s = jnp.sum(g * pred, axis=1)
    term = jnp.log(1.0 - s)
    out_ref[...] = ce_ref[...] + LAM * jnp.reshape(jnp.mean(term), (1, 1))


_stage3 = pl.pallas_call(
    _stage3_body,
    out_shape=jax.ShapeDtypeStruct((1, 1), jnp.float32),
)


def kernel(indices, output, label, stored_targets):
    label2 = label.reshape(BATCH, 1).astype(jnp.int32)
    lo_rows, st_rows = _stage2(indices, output, stored_targets)
    pred, ce = _stage1(output, label2)
    loss = _stage3(pred, lo_rows, st_rows, ce)
    return loss.reshape(())


# speedup vs baseline: 3.7012x; 1.0377x over previous
import functools

import jax
import jax.numpy as jnp
from jax import lax
from jax.experimental import pallas as pl
from jax.experimental.pallas import tpu as pltpu
from jax.experimental.pallas import tpu_sc as plsc

N_EXAMPLES = 100000
N_CLASSES = 128
BATCH = 4096
BETA = 0.3
LAM = 3.0

NC = 2
NS = 16
NW = NC * NS
ROWS_PER_TILE = BATCH // NW
SUB = 32
NSUB = ROWS_PER_TILE // SUB
NCHUNK = BATCH // 16
GRID = 4
GB = BATCH // GRID


def _stage1_body(out_ref, label_ref, pred_ref, ce_ref):
    x = out_ref[...]
    m = jnp.max(x, axis=1, keepdims=True)
    e = jnp.exp(x - m)
    se = jnp.sum(e, axis=1, keepdims=True)
    logp = (x - m) - jnp.log(se)
    pred_ref[...] = jnp.clip(e * (1.0 / se), 0.0001, 1.0 - 0.0001)
    lab = label_ref[...]
    cols = lax.broadcasted_iota(jnp.int32, x.shape, 1)
    sel = jnp.where(cols == lab, logp, 0.0)

    @pl.when(pl.program_id(0) == 0)
    def _():
        ce_ref[...] = jnp.zeros_like(ce_ref)

    ce_ref[...] += jnp.reshape(-jnp.sum(sel) / BATCH, (1, 1))


_stage1 = pl.pallas_call(
    _stage1_body,
    grid=(GRID,),
    in_specs=[
        pl.BlockSpec((GB, N_CLASSES), lambda i: (i, 0)),
        pl.BlockSpec((GB, 1), lambda i: (i, 0)),
    ],
    out_specs=(
        pl.BlockSpec((GB, N_CLASSES), lambda i: (i, 0)),
        pl.BlockSpec((1, 1), lambda i: (0, 0)),
    ),
    out_shape=(
        jax.ShapeDtypeStruct((BATCH, N_CLASSES), jnp.float32),
        jax.ShapeDtypeStruct((1, 1), jnp.float32),
    ),
)


UNROLL = 4


def _sc_body(idx_hbm, logits_hbm, stored_hbm, lo_rows_hbm, st_rows_hbm,
             table, idxv, myidx, wv, lobuf, stbuf,
             sem_sg, sem_sw, gl0, gl1, wl0, wl1):
    wid = lax.axis_index("s") * NC + lax.axis_index("c")
    base = wid * ROWS_PER_TILE

    pltpu.sync_copy(idx_hbm, idxv)

    for c in range(ROWS_PER_TILE // 16):
        myidx[pl.ds(c * 16, 16)] = idxv[pl.ds(base + c * 16, 16)]
    st_g = pltpu.async_copy(stored_hbm.at[myidx], stbuf, sem_sg)

    def chunk_body(i, carry):
        for u in range(UNROLL):
            c = i * UNROLL + u
            off = pl.multiple_of(c * 16, 16)
            keys = idxv[pl.ds(off, 16)]
            pos = jnp.full((16,), c * 16, jnp.int32) + lax.iota(jnp.int32, 16)
            _, last = plsc.scan_count(keys)
            plsc.store_scatter(table, [keys], pos, mask=last)
        return carry

    lax.fori_loop(0, NCHUNK // UNROLL, chunk_body, 0)

    st_g.wait()
    st_w = pltpu.async_copy(stbuf, st_rows_hbm.at[pl.ds(base, ROWS_PER_TILE)],
                            sem_sw)

    for c in range(ROWS_PER_TILE // 16):
        keys = myidx[pl.ds(c * 16, 16)]
        w = plsc.load_gather(table, [keys])
        b, r = (c * 16) // SUB, (c * 16) % SUB
        wv[b, pl.ds(r, 16)] = w

    gsem = (gl0, gl1)
    wsem = (wl0, wl1)
    gathers = [None, None]
    writes = [None, None]

    def start(b):
        k = b % 2
        gathers[k] = pltpu.async_copy(logits_hbm.at[wv.at[b]], lobuf.at[k],
                                      gsem[k])

    start(0)
    for b in range(NSUB):
        if b + 1 < NSUB:
            if writes[(b + 1) % 2] is not None:
                writes[(b + 1) % 2].wait()
                writes[(b + 1) % 2] = None
            start(b + 1)
        k = b % 2
        gathers[k].wait()
        writes[k] = pltpu.async_copy(
            lobuf.at[k], lo_rows_hbm.at[pl.ds(base + b * SUB, SUB)], wsem[k])
    for wcp in writes:
        if wcp is not None:
            wcp.wait()
    st_w.wait()


_stage2 = functools.partial(
    pl.kernel,
    out_type=(
        jax.ShapeDtypeStruct((BATCH, N_CLASSES), jnp.float32),
        jax.ShapeDtypeStruct((BATCH, N_CLASSES), jnp.float32),
    ),
    mesh=plsc.VectorSubcoreMesh(core_axis_name="c", subcore_axis_name="s"),
    compiler_params=pltpu.CompilerParams(needs_layout_passes=False,
                                         use_tc_tiling_on_sc=True),
    scratch_types=(
        pltpu.VMEM((N_EXAMPLES,), jnp.int32),
        pltpu.VMEM((BATCH,), jnp.int32),
        pltpu.VMEM((ROWS_PER_TILE,), jnp.int32),
        pltpu.VMEM((NSUB, SUB), jnp.int32),
        pltpu.VMEM((2, SUB, N_CLASSES), jnp.float32),
        pltpu.VMEM((ROWS_PER_TILE, N_CLASSES), jnp.float32),
        pltpu.SemaphoreType.DMA,
        pltpu.SemaphoreType.DMA,
        pltpu.SemaphoreType.DMA,
        pltpu.SemaphoreType.DMA,
        pltpu.SemaphoreType.DMA,
        pltpu.SemaphoreType.DMA,
    ),
)(_sc_body)


def _stage3_body(pred_ref, lo_rows_ref, st_rows_ref, ce_ref, out_ref):
    pred = pred_ref[...]
    x = lo_rows_ref[...]
    m = jnp.max(x, axis=1, keepdims=True)
    e = jnp.exp(x - m)
    pw = jnp.clip(e * (1.0 / jnp.sum(e, axis=1, keepdims=True)),
                  0.0001, 1.0 - 0.0001)
    pn_rows = pw * (1.0 / jnp.sum(pw, axis=1, keepdims=True))
    g = BETA * st_rows_ref[...] + (1.0 - BETA) * pn_rows
    s = jnp.sum(g * pred, axis=1)
    term = jnp.log(1.0 - s)

    @pl.when(pl.program_id(0) == 0)
    def _():
        out_ref[...] = ce_ref[...]

    out_ref[...] += LAM * jnp.reshape(jnp.sum(term) / BATCH, (1, 1))


_stage3 = pl.pallas_call(
    _stage3_body,
    grid=(GRID,),
    in_specs=[
        pl.BlockSpec((GB, N_CLASSES), lambda i: (i, 0)),
        pl.BlockSpec((GB, N_CLASSES), lambda i: (i, 0)),
        pl.BlockSpec((GB, N_CLASSES), lambda i: (i, 0)),
        pl.BlockSpec((1, 1), lambda i: (0, 0)),
    ],
    out_specs=pl.BlockSpec((1, 1), lambda i: (0, 0)),
    out_shape=jax.ShapeDtypeStruct((1, 1), jnp.float32),
)


def kernel(indices, output, label, stored_targets):
    label2 = label.reshape(BATCH, 1).astype(jnp.int32)
    lo_rows, st_rows = _stage2(indices, output, stored_targets)
    pred, ce = _stage1(output, label2)
    loss = _stage3(pred, lo_rows, st_rows, ce)
    return loss.reshape(())
